# Initial kernel scaffold; baseline (speedup 1.0000x reference)
import jax
import jax.numpy as jnp
from jax.experimental import pallas as pl


def kernel(x, edge_index, edge_attr, W0, b0, Wc1, as1, ad1, We1, ae1, bc1, Wc2, as2, ad2, We2, ae2, bc2, W1, b1, W2, b2):
    raise NotImplementedError("write your pallas kernel here")


# speedup vs baseline: 12.1837x; 12.1837x over previous
import functools

import jax
import jax.numpy as jnp
from jax import lax
from jax.experimental import pallas as pl
from jax.experimental.pallas import tpu as pltpu
from jax.experimental.pallas import tpu_sc as plsc

N = 10000
E = 320000
DF = 128
DE = 16
H = 64
HP = 128
NC = 2
NS = 16
NW = NC * NS
CE = 128
NCHUNK = E // CE
NP = 10240
RPT = NP // NS
RQ = CE

_sc_mesh = plsc.VectorSubcoreMesh(core_axis_name="c", subcore_axis_name="s")
_sc_params = pltpu.CompilerParams(needs_layout_passes=False)


@functools.partial(
    pl.kernel,
    out_type=jax.ShapeDtypeStruct((NC, NP, HP), jnp.float32),
    mesh=_sc_mesh,
    compiler_params=_sc_params,
    scratch_types=[
        pltpu.VMEM((N,), jnp.float32),
        pltpu.VMEM((N,), jnp.float32),
        pltpu.VMEM((CE,), jnp.int32),
        pltpu.VMEM((CE,), jnp.int32),
        pltpu.VMEM((CE,), jnp.float32),
        pltpu.VMEM((CE,), jnp.float32),
        pltpu.VMEM((CE, HP), jnp.float32),
        pltpu.VMEM_SHARED((NP, HP), jnp.float32),
        pltpu.SemaphoreType.DMA,
    ],
)
def _gat_edge_pass(src_hbm, dst_hbm, ae_hbm, asrc_hbm, adst_hbm, ht_hbm,
                   out_hbm, asrc_v, adst_v, src_v, dst_v, ae_v, ex_v,
                   rows_v, acc_sh, sem):
    c = lax.axis_index("c")
    s = lax.axis_index("s")
    w = s * NC + c

    def _zero_body(r, carry):
        for q in range(HP // 16):
            rows_v[r, pl.ds(q * 16, 16)] = jnp.zeros((16,), jnp.float32)
        return carry
    lax.fori_loop(0, CE, _zero_body, 0)
    for j in range(RPT // RQ):
        r0 = s * RPT + j * RQ
        pltpu.sync_copy(rows_v, acc_sh.at[pl.ds(r0, RQ)])

    pltpu.sync_copy(asrc_hbm, asrc_v)
    pltpu.sync_copy(adst_hbm, adst_v)
    plsc.subcore_barrier()

    nchunks = (NCHUNK - w + NW - 1) // NW

    def _chunk_body(j, carry):
        base = (w + j * NW) * CE
        pltpu.sync_copy(src_hbm.at[pl.ds(base, CE)], src_v)
        gat = pltpu.async_copy(ht_hbm.at[src_v], rows_v, sem)
        pltpu.sync_copy(dst_hbm.at[pl.ds(base, CE)], dst_v)
        pltpu.sync_copy(ae_hbm.at[pl.ds(base, CE)], ae_v)
        for g in range(CE // 16):
            sl = pl.ds(g * 16, 16)
            si = src_v[sl]
            di = dst_v[sl]
            lg = (plsc.load_gather(asrc_v, [si])
                  + plsc.load_gather(adst_v, [di]) + ae_v[sl])
            lg = jnp.where(lg > 0, lg, 0.2 * lg)
            ex_v[sl] = jnp.exp(lg)
        gat.wait()

        def _scale_body(e, carry2):
            m = plsc.load_gather(ex_v, [jnp.full((16,), e, jnp.int32)])
            for q in range(HP // 16):
                sl2 = pl.ds(q * 16, 16)
                rows_v[e, sl2] = rows_v[e, sl2] * m
            return carry2
        lax.fori_loop(0, CE, _scale_body, 0)

        pltpu.sync_copy(rows_v, acc_sh.at[dst_v], add=True)
        return carry

    lax.fori_loop(0, nchunks, _chunk_body, 0)
    plsc.subcore_barrier()

    for j in range(RPT // RQ):
        r0 = s * RPT + j * RQ
        pltpu.sync_copy(acc_sh.at[pl.ds(r0, RQ)], out_hbm.at[c, pl.ds(r0, RQ)])


@functools.partial(
    pl.kernel,
    out_type=jax.ShapeDtypeStruct((E,), jnp.float32),
    mesh=_sc_mesh,
    compiler_params=_sc_params,
    scratch_types=[
        pltpu.VMEM((CE,), jnp.int32),
        pltpu.VMEM((CE,), jnp.int32),
        pltpu.VMEM((CE, DF), jnp.float32),
        pltpu.VMEM((CE, DF), jnp.float32),
        pltpu.VMEM((CE,), jnp.float32),
        pltpu.VMEM((DF,), jnp.float32),
        pltpu.SemaphoreType.DMA,
        pltpu.SemaphoreType.DMA,
    ],
)
def _decode_pass(src_hbm, dst_hbm, uv_hbm, wp_hbm, out_hbm,
                 src_v, dst_v, urows_v, vrows_v, out_v, wp_v, semu, semv):
    c = lax.axis_index("c")
    s = lax.axis_index("s")
    w = s * NC + c
    pltpu.sync_copy(wp_hbm, wp_v)
    nchunks = (NCHUNK - w + NW - 1) // NW

    def _chunk_body(j, carry):
        base = (w + j * NW) * CE
        pltpu.sync_copy(src_hbm.at[pl.ds(base, CE)], src_v)
        gu = pltpu.async_copy(uv_hbm.at[src_v], urows_v, semu)
        pltpu.sync_copy(dst_hbm.at[pl.ds(base, CE)], dst_v)
        gv = pltpu.async_copy(uv_hbm.at[dst_v], vrows_v, semv)
        gu.wait()
        gv.wait()
        lanes = lax.iota(jnp.int32, 16)
        for g in range(CE // 16):
            rowi = lanes + g * 16

            def _feat_body(k4, acc):
                for u in range(4):
                    k = k4 * 4 + u
                    ck = jnp.full((16,), k, jnp.int32)
                    uk = plsc.load_gather(urows_v, [rowi, ck])
                    vk = plsc.load_gather(vrows_v, [rowi, ck + H])
                    wk = plsc.load_gather(wp_v, [ck])
                    acc = acc + jnp.maximum(uk + vk, 0.0) * wk
                return acc

            acc = lax.fori_loop(0, H // 4, _feat_body,
                                jnp.zeros((16,), jnp.float32))
            lg = acc + plsc.load_gather(
                wp_v, [jnp.full((16,), H, jnp.int32)])
            out_v[pl.ds(g * 16, 16)] = 1.0 / (1.0 + jnp.exp(-lg))
        pltpu.sync_copy(out_v, out_hbm.at[pl.ds(base, CE)])
        return carry

    lax.fori_loop(0, nchunks, _chunk_body, 0)


BN = 2000


def _tc_encode_body(x_ref, W0_ref, b0_ref, Wc_ref, as_ref, ad_ref,
                    ht_ref, asrc_ref, adst_ref):
    z = jnp.dot(x_ref[...], W0_ref[...],
                preferred_element_type=jnp.float32) + b0_ref[...]
    h = jnp.dot(z, Wc_ref[...], preferred_element_type=jnp.float32)
    pad = jnp.concatenate(
        [h, jnp.ones((h.shape[0], 1), jnp.float32),
         jnp.zeros((h.shape[0], HP - H - 1), jnp.float32)], axis=1)
    ht_ref[...] = pad
    asrc_ref[...] = jnp.sum(h * as_ref[...], axis=1, keepdims=True)
    adst_ref[...] = jnp.sum(h * ad_ref[...], axis=1, keepdims=True)


def _tc_mid_body(acc_ref, bc_ref, Wc_ref, as_ref, ad_ref,
                 ht_ref, asrc_ref, adst_ref):
    a = acc_ref[0] + acc_ref[1]
    den = a[:, H:H + 1]
    z = jnp.maximum(a[:, :H] / (den + 1e-16) + bc_ref[...], 0.0)
    h = jnp.dot(z, Wc_ref[...], preferred_element_type=jnp.float32)
    pad = jnp.concatenate(
        [h, jnp.ones((h.shape[0], 1), jnp.float32),
         jnp.zeros((h.shape[0], HP - H - 1), jnp.float32)], axis=1)
    ht_ref[...] = pad
    asrc_ref[...] = jnp.sum(h * as_ref[...], axis=1, keepdims=True)
    adst_ref[...] = jnp.sum(h * ad_ref[...], axis=1, keepdims=True)


def _tc_final_body(acc_ref, bc_ref, W1t_ref, W1b_ref, b1_ref, uv_ref):
    a = acc_ref[0] + acc_ref[1]
    den = a[:, H:H + 1]
    z = jnp.maximum(a[:, :H] / (den + 1e-16) + bc_ref[...], 0.0)
    u = jnp.dot(z, W1t_ref[...],
                preferred_element_type=jnp.float32) + b1_ref[...]
    v = jnp.dot(z, W1b_ref[...], preferred_element_type=jnp.float32)
    uv_ref[...] = jnp.concatenate([u, v], axis=1)


def _tc_ae_body(ea_ref, Wd_ref, out_ref):
    out_ref[...] = jnp.dot(ea_ref[...], Wd_ref[...],
                           preferred_element_type=jnp.float32)


def _row_spec(bn, ncols):
    return pl.BlockSpec((bn, ncols), lambda i: (i, 0))


def _full_spec(shape):
    return pl.BlockSpec(shape, lambda i: tuple(0 for _ in shape))


def _tc_encode(x, W0, b0r, Wc, asr, adr):
    grid = (N // BN,)
    return pl.pallas_call(
        _tc_encode_body,
        grid=grid,
        in_specs=[_row_spec(BN, DF), _full_spec((DF, H)), _full_spec((1, H)),
                  _full_spec((H, H)), _full_spec((1, H)), _full_spec((1, H))],
        out_specs=[_row_spec(BN, HP),
                   _row_spec(BN, 1), _row_spec(BN, 1)],
        out_shape=[jax.ShapeDtypeStruct((N, HP), jnp.float32),
                   jax.ShapeDtypeStruct((N, 1), jnp.float32),
                   jax.ShapeDtypeStruct((N, 1), jnp.float32)],
    )(x, W0, b0r, Wc, asr, adr)


def _tc_mid(acc, bcr, Wc, asr, adr):
    grid = (N // BN,)
    return pl.pallas_call(
        _tc_mid_body,
        grid=grid,
        in_specs=[pl.BlockSpec((NC, BN, HP), lambda i: (0, i, 0)),
                  _full_spec((1, H)), _full_spec((H, H)),
                  _full_spec((1, H)), _full_spec((1, H))],
        out_specs=[_row_spec(BN, HP),
                   _row_spec(BN, 1), _row_spec(BN, 1)],
        out_shape=[jax.ShapeDtypeStruct((N, HP), jnp.float32),
                   jax.ShapeDtypeStruct((N, 1), jnp.float32),
                   jax.ShapeDtypeStruct((N, 1), jnp.float32)],
    )(acc, bcr, Wc, asr, adr)


def _tc_final(acc, bcr, W1t, W1b, b1r):
    grid = (N // BN,)
    return pl.pallas_call(
        _tc_final_body,
        grid=grid,
        in_specs=[pl.BlockSpec((NC, BN, HP), lambda i: (0, i, 0)),
                  _full_spec((1, H)), _full_spec((H, H)),
                  _full_spec((H, H)), _full_spec((1, H))],
        out_specs=_row_spec(BN, DF),
        out_shape=jax.ShapeDtypeStruct((N, DF), jnp.float32),
    )(acc, bcr, W1t, W1b, b1r)


def _tc_ae(ea128, Wd):
    R = E // 8
    BR = R // 5
    return pl.pallas_call(
        _tc_ae_body,
        grid=(5,),
        in_specs=[_row_spec(BR, DF), _full_spec((DF, 16))],
        out_specs=_row_spec(BR, 16),
        out_shape=jax.ShapeDtypeStruct((R, 16), jnp.float32),
    )(ea128, Wd)


def kernel(x, edge_index, edge_attr, W0, b0, Wc1, as1, ad1, We1, ae1, bc1,
           Wc2, as2, ad2, We2, ae2, bc2, W1, b1, W2, b2):
    src = edge_index[0]
    dst = edge_index[1]

    b0r = b0.reshape(1, H)
    bc1r = bc1.reshape(1, H)
    bc2r = bc2.reshape(1, H)
    b1r = b1.reshape(1, H)
    as1r = as1.reshape(1, H)
    ad1r = ad1.reshape(1, H)
    as2r = as2.reshape(1, H)
    ad2r = ad2.reshape(1, H)
    w1e = We1 @ ae1
    w2e = We2 @ ae2
    rows = jnp.arange(DF)
    arow = rows // DE
    krow = rows % DE
    Wd = jnp.zeros((DF, 16), jnp.float32)
    Wd = Wd.at[rows, arow].set(w1e[krow])
    Wd = Wd.at[rows, 8 + arow].set(w2e[krow])
    ea128 = edge_attr.reshape(E // 8, DF)
    W1t = W1[:H]
    W1b = W1[H:]
    wp = jnp.concatenate(
        [W2.reshape(H), b2.reshape(1), jnp.zeros((DF - H - 1,), jnp.float32)])

    ht1, asrc1, adst1 = _tc_encode(x, W0, b0r, Wc1, as1r, ad1r)
    aeo = _tc_ae(ea128, Wd)
    ae1v = aeo[:, :8].reshape(E)
    ae2v = aeo[:, 8:].reshape(E)

    acc1 = _gat_edge_pass(src, dst, ae1v, asrc1.reshape(N),
                          adst1.reshape(N), ht1)
    ht2, asrc2, adst2 = _tc_mid(acc1, bc1r, Wc2, as2r, ad2r)
    acc2 = _gat_edge_pass(src, dst, ae2v, asrc2.reshape(N),
                          adst2.reshape(N), ht2)
    uv = _tc_final(acc2, bc2r, W1t, W1b, b1r)
    return _decode_pass(src, dst, uv, wp)
